# baseline (speedup 1.0000x reference)
import jax
import jax.numpy as jnp
from jax.experimental import pallas as pl


def _sage(x_src, x_dst, ei, Wl, bl, Wr, n_dst):
    src, dst = ei[0], ei[1]
    msg = x_src[src]
    s = jax.ops.segment_sum(msg, dst, num_segments=n_dst)
    cnt = jax.ops.segment_sum(jnp.ones((ei.shape[1],), x_src.dtype), dst, num_segments=n_dst)
    mean = s / jnp.clip(cnt, 1.0)[:, None]
    return mean @ Wl.T + bl + x_dst @ Wr.T


def _hetero(xu, xa, xl, es, Wl, bl, Wr, NU, NA, NL):
    o0 = _sage(xu, xa, es[0], Wl[0], bl[0], Wr[0], NA)
    o1 = _sage(xu, xa, es[1], Wl[1], bl[1], Wr[1], NA)
    o2 = _sage(xa, xu, es[2], Wl[2], bl[2], Wr[2], NU)
    o3 = _sage(xa, xu, es[3], Wl[3], bl[3], Wr[3], NU)
    o4 = _sage(xa, xl, es[4], Wl[4], bl[4], Wr[4], NL)
    o5 = _sage(xl, xa, es[5], Wl[5], bl[5], Wr[5], NA)
    return (o2 + o3) / 2.0, (o0 + o1 + o5) / 3.0, o4


def _identity_kernel(x_ref, o_ref):
    o_ref[...] = x_ref[...]


def kernel(x_user, x_ad, x_location, e_click, e_purchase, e_rev_click, e_rev_purchase, e_in, e_contains, Wl1, bl1, Wr1, Wl2, bl2, Wr2, Wl3, bl3, Wr3, W1, b1, W2, b2):
    NU, NA, NL = x_user.shape[0], x_ad.shape[0], x_location.shape[0]
    es = (e_click, e_purchase, e_rev_click, e_rev_purchase, e_in, e_contains)
    xu, xa, xl = _hetero(x_user, x_ad, x_location, es, Wl1, bl1, Wr1, NU, NA, NL)
    xu, xa, xl = jax.nn.relu(xu), jax.nn.relu(xa), jax.nn.relu(xl)
    xu, xa, xl = _hetero(xu, xa, xl, es, Wl2, bl2, Wr2, NU, NA, NL)
    xu, xa, xl = jax.nn.relu(xu), jax.nn.relu(xa), jax.nn.relu(xl)
    xu, xa, xl = _hetero(xu, xa, xl, es, Wl3, bl3, Wr3, NU, NA, NL)
    xu, xa, xl = jax.nn.relu(xu), jax.nn.relu(xa), jax.nn.relu(xl)
    xu = jax.nn.relu(xu @ W1.T + b1) @ W2.T + b2
    xa = jax.nn.relu(xa @ W1.T + b1) @ W2.T + b2
    xl = jax.nn.relu(xl @ W1.T + b1) @ W2.T + b2
    xu = pl.pallas_call(
        _identity_kernel,
        out_shape=jax.ShapeDtypeStruct(xu.shape, xu.dtype),
    )(xu)
    return (xu, xa, xl)


# speedup vs baseline: 1.1993x; 1.1993x over previous
import jax
import jax.numpy as jnp
from jax import lax
from jax.experimental import pallas as pl
from jax.experimental.pallas import tpu as pltpu
from jax.experimental.pallas import tpu_sc as plsc

NU, NA, NL, D, H = 50000, 50000, 10000, 128, 128
E_BIG, E_SMALL = 300000, 100000

W = 16
NCG = D // W
B = 128
SENTINEL = 1 << 20

REL_SRC = ("u", "u", "a", "a", "a", "l")
REL_DST = ("a", "a", "u", "u", "l", "a")
NDST = {"u": NU, "a": NA, "l": NL}
REL_E = (E_BIG, E_BIG, E_BIG, E_BIG, E_SMALL, E_SMALL)

SC_RELS = ((0, 1, 5), (2, 3, 4))


def _round_up(x, m):
    return (x + m - 1) // m * m


_EP = {E_BIG: _round_up(E_BIG, 16 * B), E_SMALL: _round_up(E_SMALL, 16 * B)}
_WSL = {e: p // 16 for e, p in _EP.items()}
REL_NDS = tuple(_round_up(NDST[t] + 16, 16) for t in REL_DST)
ACC_ROWS = max(REL_NDS) + 16

_mesh = plsc.VectorSubcoreMesh(core_axis_name="c", subcore_axis_name="s")
_cparams = pltpu.CompilerParams(use_tc_tiling_on_sc=False)


def _iota16():
    return lax.iota(jnp.int32, 16)


def _count_body(*refs):
    dsts = refs[0:6]
    ones_h, zeros_h = refs[6], refs[7]
    outs = refs[8:14]
    dvm, idxb, onesb, acc, sem = refs[14:]
    core = lax.axis_index("c")
    s = lax.axis_index("s")
    iota = _iota16()
    pltpu.sync_copy(ones_h, onesb)

    for my_core in (0, 1):
        @pl.when(core == my_core)
        def _():
            for r in SC_RELS[my_core]:
                nds = REL_NDS[r]
                wsl = _WSL[REL_E[r]]
                nz = (nds + 16) // 16
                pltpu.sync_copy(zeros_h.at[pl.ds(s * nz, nz)],
                                acc.at[pl.ds(s * nz, nz)])
                plsc.subcore_barrier()
                pltpu.sync_copy(dsts[r].at[pl.ds(s * wsl, wsl)],
                                dvm.at[pl.ds(0, wsl)])
                garb = nds + lax.bitwise_and(iota, 7)

                def batch(t, _):
                    for u in range(8):
                        dv = dvm[pl.ds(t * B + 16 * u, 16)]
                        idxb[pl.ds(16 * u, 16)] = jnp.minimum(dv, garb)
                    pltpu.sync_copy(onesb, acc.at[idxb], add=True)
                    return 0
                lax.fori_loop(0, wsl // B, batch, 0)
                plsc.subcore_barrier()
                nr = nds // 16
                pltpu.sync_copy(acc.at[pl.ds(s * nr, nr)],
                                outs[r].at[pl.ds(s * nr, nr)])
                plsc.subcore_barrier()


def _run_counts(dsts):
    out_type = tuple(jax.ShapeDtypeStruct((REL_NDS[r], W), jnp.float32)
                     for r in range(6))
    scratch = [
        pltpu.VMEM((_WSL[E_BIG],), jnp.int32),
        pltpu.VMEM((B,), jnp.int32),
        pltpu.VMEM((B, W), jnp.float32),
        pltpu.VMEM_SHARED((ACC_ROWS, W), jnp.float32),
        pltpu.SemaphoreType.DMA,
    ]
    ones = jnp.ones((B, W), jnp.float32)
    zeros = jnp.zeros((ACC_ROWS, W), jnp.float32)
    fn = pl.kernel(_count_body, out_type=out_type, mesh=_mesh,
                   scratch_types=scratch, compiler_params=_cparams)
    return fn(*dsts, ones, zeros)


def _segsum_body(*refs):
    srcs = refs[0:6]
    dsts = refs[6:12]
    tabs = {"u": refs[12:20], "a": refs[20:28], "l": refs[28:36]}
    zeros_h = refs[36]
    outs = refs[37:43]
    svm, dvm, dstb, rowsb, acc, gsem = refs[43:]
    core = lax.axis_index("c")
    s = lax.axis_index("s")
    iota = _iota16()

    for my_core in (0, 1):
        @pl.when(core == my_core)
        def _():
            for r in SC_RELS[my_core]:
                nds = REL_NDS[r]
                wsl = _WSL[REL_E[r]]
                pltpu.sync_copy(srcs[r].at[pl.ds(s * wsl, wsl)],
                                svm.at[pl.ds(0, wsl)])
                pltpu.sync_copy(dsts[r].at[pl.ds(s * wsl, wsl)],
                                dvm.at[pl.ds(0, wsl)])
                garb = nds + lax.bitwise_and(iota, 7)
                nz = (nds + 16) // 16
                nr = nds // 16
                for cg in range(NCG):
                    tab = tabs[REL_SRC[r]][cg]
                    pltpu.sync_copy(zeros_h.at[pl.ds(s * nz, nz)],
                                    acc.at[pl.ds(s * nz, nz)])
                    plsc.subcore_barrier()

                    def batch(t, _, _tab=tab, _garb=garb):
                        for u in range(8):
                            dv = dvm[pl.ds(t * B + 16 * u, 16)]
                            dstb[pl.ds(16 * u, 16)] = jnp.minimum(dv, _garb)
                        pltpu.async_copy(
                            _tab.at[svm.at[pl.ds(t * B, B)]], rowsb, gsem
                        ).wait()
                        pltpu.sync_copy(rowsb, acc.at[dstb], add=True)
                        return 0
                    lax.fori_loop(0, wsl // B, batch, 0)
                    plsc.subcore_barrier()
                    pltpu.sync_copy(acc.at[pl.ds(s * nr, nr)],
                                    outs[r].at[cg, pl.ds(s * nr, nr)])
                    plsc.subcore_barrier()


def _run_segsum(edges_src, edges_dst, xu8, xa8, xl8, zeros):
    out_type = tuple(jax.ShapeDtypeStruct((NCG, REL_NDS[r], W), jnp.float32)
                     for r in range(6))
    scratch = [
        pltpu.VMEM((_WSL[E_BIG],), jnp.int32),
        pltpu.VMEM((_WSL[E_BIG],), jnp.int32),
        pltpu.VMEM((B,), jnp.int32),
        pltpu.VMEM((B, W), jnp.float32),
        pltpu.VMEM_SHARED((ACC_ROWS, W), jnp.float32),
        pltpu.SemaphoreType.DMA,
    ]
    fn = pl.kernel(_segsum_body, out_type=out_type, mesh=_mesh,
                   scratch_types=scratch, compiler_params=_cparams)
    return fn(*edges_src, *edges_dst, *xu8, *xa8, *xl8, zeros)


def _dense_layer(ss, cnts, xu, xa, xl, Wl, bl, Wr):
    m = []
    for r in range(6):
        n = NDST[REL_DST[r]]
        s_full = jnp.transpose(ss[r], (1, 0, 2)).reshape(REL_NDS[r], D)[:n]
        inv = 1.0 / jnp.clip(cnts[r][:n, :1], 1.0)
        m.append(s_full * inv)
    Wa = jnp.concatenate(
        [Wl[0].T, Wl[1].T, Wl[5].T, (Wr[0] + Wr[1] + Wr[5]).T], axis=0) / 3.0
    Wu = jnp.concatenate([Wl[2].T, Wl[3].T, (Wr[2] + Wr[3]).T], axis=0) / 2.0
    Wlo = jnp.concatenate([Wl[4].T, Wr[4].T], axis=0)
    ba = (bl[0] + bl[1] + bl[5]) / 3.0
    bu = (bl[2] + bl[3]) / 2.0
    xa_n = jnp.concatenate([m[0], m[1], m[5], xa], axis=1) @ Wa + ba
    xu_n = jnp.concatenate([m[2], m[3], xu], axis=1) @ Wu + bu
    xl_n = jnp.concatenate([m[4], xl], axis=1) @ Wlo + bl[4]
    return jax.nn.relu(xu_n), jax.nn.relu(xa_n), jax.nn.relu(xl_n)


def _split16(x):
    return [x[:, W * g:W * (g + 1)] for g in range(NCG)]


def kernel(x_user, x_ad, x_location, e_click, e_purchase, e_rev_click,
           e_rev_purchase, e_in, e_contains, Wl1, bl1, Wr1, Wl2, bl2, Wr2,
           Wl3, bl3, Wr3, W1, b1, W2, b2):
    edges = (e_click, e_purchase, e_rev_click, e_rev_purchase, e_in, e_contains)
    esrc, edst = [], []
    for r, e in enumerate(edges):
        ep = _EP[REL_E[r]]
        pad = ep - REL_E[r]
        esrc.append(jnp.concatenate([e[0], jnp.zeros((pad,), jnp.int32)]))
        edst.append(jnp.concatenate(
            [e[1], jnp.full((pad,), SENTINEL, jnp.int32)]))

    cnts = _run_counts(edst)
    zeros = jnp.zeros((ACC_ROWS, W), jnp.float32)

    xu, xa, xl = x_user, x_ad, x_location
    for (Wl, bl, Wr) in ((Wl1, bl1, Wr1), (Wl2, bl2, Wr2), (Wl3, bl3, Wr3)):
        ss = _run_segsum(esrc, edst, _split16(xu), _split16(xa), _split16(xl),
                         zeros)
        xu, xa, xl = _dense_layer(ss, cnts, xu, xa, xl, Wl, bl, Wr)

    xu = jax.nn.relu(xu @ W1.T + b1) @ W2.T + b2
    xa = jax.nn.relu(xa @ W1.T + b1) @ W2.T + b2
    xl = jax.nn.relu(xl @ W1.T + b1) @ W2.T + b2
    return (xu, xa, xl)


# speedup vs baseline: 1.2708x; 1.0596x over previous
import numpy as np

import jax
import jax.numpy as jnp
from jax import lax
from jax.experimental import pallas as pl
from jax.experimental.pallas import tpu as pltpu
from jax.experimental.pallas import tpu_sc as plsc

NU, NA, NL, D, H = 50000, 50000, 10000, 128, 128
E_BIG, E_SMALL = 300000, 100000

W = 16
NCG = D // W
B = 128
SENTINEL = 1 << 20

REL_SRC = ("u", "u", "a", "a", "a", "l")
REL_DST = ("a", "a", "u", "u", "l", "a")
NDST = {"u": NU, "a": NA, "l": NL}
REL_E = (E_BIG, E_BIG, E_BIG, E_BIG, E_SMALL, E_SMALL)

SC_RELS = ((0, 1, 5), (2, 3, 4))


def _round_up(x, m):
    return (x + m - 1) // m * m


_EP = {e: _round_up(e, 16 * B * 4) for e in (E_BIG, E_SMALL)}
_WSL = {e: p // 16 for e, p in _EP.items()}
EPOFF = (0, _EP[E_BIG], 2 * _EP[E_BIG], 3 * _EP[E_BIG],
         4 * _EP[E_BIG], 4 * _EP[E_BIG] + _EP[E_SMALL])
E_TOTAL = 4 * _EP[E_BIG] + 2 * _EP[E_SMALL] + 16384

TABBASE = {"u": 0, "a": NCG * NU, "l": NCG * (NU + NA)}
TAB_ROWS = NCG * (NU + NA + NL)

SEG_NDS = tuple(_round_up(NDST[t] + 16, 16 * B) for t in REL_DST)
REGOFF = tuple(int(x) for x in np.cumsum((0,) + tuple(
    NCG * n for n in SEG_NDS))[:6])
S_ROWS = REGOFF[5] + NCG * SEG_NDS[5]
ACC_ROWS = max(SEG_NDS) + 16

CNT_NDS = tuple(_round_up(NDST[t] + 16, 16) for t in REL_DST)

_mesh = plsc.VectorSubcoreMesh(core_axis_name="c", subcore_axis_name="s")
_cparams = pltpu.CompilerParams(use_tc_tiling_on_sc=False)


def _iota16():
    return lax.iota(jnp.int32, 16)


def _sel3(i8, vals):
    return jnp.where(i8 == 0, jnp.int32(vals[0]),
                     jnp.where(i8 == 1, jnp.int32(vals[1]),
                               jnp.int32(vals[2])))


def _count_body(*refs):
    dsts = refs[0:6]
    ones_h, zeros_h = refs[6], refs[7]
    outs = refs[8:14]
    dvm, idxb, onesb, acc, sem = refs[14:]
    core = lax.axis_index("c")
    s = lax.axis_index("s")
    iota = _iota16()
    pltpu.sync_copy(ones_h, onesb)

    for my_core in (0, 1):
        @pl.when(core == my_core)
        def _():
            for r in SC_RELS[my_core]:
                nds = CNT_NDS[r]
                wsl = _WSL[REL_E[r]]
                nz = (nds + 16) // 16
                pltpu.sync_copy(zeros_h.at[pl.ds(s * nz, nz)],
                                acc.at[pl.ds(s * nz, nz)])
                plsc.subcore_barrier()
                pltpu.sync_copy(dsts[r].at[pl.ds(s * wsl, wsl)],
                                dvm.at[pl.ds(0, wsl)])
                garb = nds + lax.bitwise_and(iota, 7)

                def batch(t, _):
                    for u in range(8):
                        dv = dvm[pl.ds(t * B + 16 * u, 16)]
                        idxb[pl.ds(16 * u, 16)] = jnp.minimum(dv, garb)
                    pltpu.sync_copy(onesb, acc.at[idxb], add=True)
                    return 0
                lax.fori_loop(0, wsl // B, batch, 0)
                plsc.subcore_barrier()
                nr = nds // 16
                pltpu.sync_copy(acc.at[pl.ds(s * nr, nr)],
                                outs[r].at[pl.ds(s * nr, nr)])
                plsc.subcore_barrier()


def _run_counts(dsts):
    out_type = tuple(jax.ShapeDtypeStruct((CNT_NDS[r], W), jnp.float32)
                     for r in range(6))
    scratch = [
        pltpu.VMEM((_WSL[E_BIG],), jnp.int32),
        pltpu.VMEM((B,), jnp.int32),
        pltpu.VMEM((B, W), jnp.float32),
        pltpu.VMEM_SHARED((max(CNT_NDS) + 32, W), jnp.float32),
        pltpu.SemaphoreType.DMA,
    ]
    ones = jnp.ones((B, W), jnp.float32)
    zeros = jnp.zeros((max(CNT_NDS) + 32, W), jnp.float32)
    fn = pl.kernel(_count_body, out_type=out_type, mesh=_mesh,
                   scratch_types=scratch, compiler_params=_cparams)
    return fn(*dsts, ones, zeros)


def _make_offtab():
    t = np.zeros((48, 2, 16), np.int32)
    for r in range(6):
        nsrc = NDST[REL_SRC[r]]
        for cg in range(NCG):
            t[r * 8 + cg, 0, :] = TABBASE[REL_SRC[r]] + cg * nsrc
            t[r * 8 + cg, 1, :] = SEG_NDS[r] + (np.arange(16) & 7)
    return t


_OFFTAB = _make_offtab()


def _segsum_body(asrc, adst, btab, offtab_h, zeros_h, sout, svm, dvm, offb,
                 i0b, i1b, i2b, i3b, d0b, d1b, d2b, d3b, r0b, r1b, r2b, r3b,
                 g0s, g1s, g2s, g3s, s0s, s1s, s2s, s3s, acc_ref):
    idxbs = (i0b, i1b, i2b, i3b)
    dstbs = (d0b, d1b, d2b, d3b)
    rows = (r0b, r1b, r2b, r3b)
    gsems = (g0s, g1s, g2s, g3s)
    ssems = (s0s, s1s, s2s, s3s)
    core = lax.axis_index("c")
    s = lax.axis_index("s")

    for my_core in (0, 1):
        rels = SC_RELS[my_core]

        @pl.when(core == my_core)
        def _():
            def pass_body(i, _):
                i8 = lax.div(i, 8)
                cg = lax.rem(i, 8)
                epoff = _sel3(i8, [EPOFF[r] for r in rels])
                ng = _sel3(i8, [_WSL[REL_E[r]] // B // 4 for r in rels])
                nds = _sel3(i8, [SEG_NDS[r] for r in rels])
                nrg = _sel3(i8, [SEG_NDS[r] // 16 // B for r in rels])
                regoff = _sel3(i8, [REGOFF[r] for r in rels])
                offrow = _sel3(i8, [r * 8 for r in rels]) + cg
                nr = nds // 16
                wslb = _WSL[E_BIG]

                @pl.when(cg == 0)
                def _():
                    woff = epoff + s * _sel3(i8, [_WSL[REL_E[r]] for r in rels])
                    pltpu.sync_copy(asrc.at[pl.ds(woff, wslb)],
                                    svm.at[pl.ds(0, wslb)])
                    pltpu.sync_copy(adst.at[pl.ds(woff, wslb)],
                                    dvm.at[pl.ds(0, wslb)])

                pltpu.sync_copy(offtab_h.at[offrow], offb)
                offv = offb[0]
                garbv = offb[1]

                def zloop(j, _):
                    pltpu.sync_copy(zeros_h,
                                    acc_ref.at[pl.ds(s * nr + j * B, B)])
                    return 0
                lax.fori_loop(0, nrg, zloop, 0)
                plsc.subcore_barrier()

                def build(b, t):
                    for u in range(8):
                        base = t * B + 16 * u
                        sv = svm[pl.ds(base, 16)]
                        dv = dvm[pl.ds(base, 16)]
                        idxbs[b][pl.ds(16 * u, 16)] = sv + offv
                        dstbs[b][pl.ds(16 * u, 16)] = jnp.minimum(dv, garbv)

                for b in range(4):
                    build(b, jnp.int32(b))
                    pltpu.async_copy(btab.at[idxbs[b]], rows[b], gsems[b])
                for b in range(4):
                    pltpu.make_async_copy(btab.at[idxbs[b]], rows[b],
                                          gsems[b]).wait()
                    pltpu.async_copy(rows[b], acc_ref.at[dstbs[b]], ssems[b],
                                     add=True)

                def grp(k, _):
                    for b in range(4):
                        pltpu.make_async_copy(rows[b], acc_ref.at[dstbs[b]],
                                              ssems[b]).wait()
                        build(b, k * 4 + b)
                        pltpu.async_copy(btab.at[idxbs[b]], rows[b], gsems[b])
                    for b in range(4):
                        pltpu.make_async_copy(btab.at[idxbs[b]], rows[b],
                                              gsems[b]).wait()
                        pltpu.async_copy(rows[b], acc_ref.at[dstbs[b]],
                                         ssems[b], add=True)
                    return 0
                lax.fori_loop(1, ng, grp, 0)
                for b in range(4):
                    pltpu.make_async_copy(rows[b], acc_ref.at[dstbs[b]],
                                          ssems[b]).wait()
                plsc.subcore_barrier()

                fbase = regoff + cg * nds + s * nr

                def floop(j, _):
                    pltpu.sync_copy(acc_ref.at[pl.ds(s * nr + j * B, B)],
                                    sout.at[pl.ds(fbase + j * B, B)])
                    return 0
                lax.fori_loop(0, nrg, floop, 0)
                plsc.subcore_barrier()
                return 0

            lax.fori_loop(0, 24, pass_body, 0)


def _run_segsum(all_src, all_dst, btab, offtab, zeros):
    out_type = jax.ShapeDtypeStruct((S_ROWS, W), jnp.float32)
    scratch = (
        [pltpu.VMEM((_WSL[E_BIG],), jnp.int32)] * 2
        + [pltpu.VMEM((2, 16), jnp.int32)]
        + [pltpu.VMEM((B,), jnp.int32) for _ in range(8)]
        + [pltpu.VMEM((B, W), jnp.float32) for _ in range(4)]
        + [pltpu.SemaphoreType.DMA for _ in range(8)]
        + [pltpu.VMEM_SHARED((ACC_ROWS, W), jnp.float32)]
    )
    fn = pl.kernel(_segsum_body, out_type=out_type, mesh=_mesh,
                   scratch_types=scratch, compiler_params=_cparams)
    return fn(all_src, all_dst, btab, offtab, zeros)


def _dense_layer(sall, cnts, xu, xa, xl, Wl, bl, Wr):
    m = []
    for r in range(6):
        n = NDST[REL_DST[r]]
        nds = SEG_NDS[r]
        s4 = lax.dynamic_slice_in_dim(sall, REGOFF[r], NCG * nds, 0)
        s_full = jnp.transpose(s4.reshape(NCG, nds, W), (1, 0, 2)
                               ).reshape(nds, D)[:n]
        inv = 1.0 / jnp.clip(cnts[r][:n, :1], 1.0)
        m.append(s_full * inv)
    Wa = jnp.concatenate(
        [Wl[0].T, Wl[1].T, Wl[5].T, (Wr[0] + Wr[1] + Wr[5]).T], axis=0) / 3.0
    Wu = jnp.concatenate([Wl[2].T, Wl[3].T, (Wr[2] + Wr[3]).T], axis=0) / 2.0
    Wlo = jnp.concatenate([Wl[4].T, Wr[4].T], axis=0)
    ba = (bl[0] + bl[1] + bl[5]) / 3.0
    bu = (bl[2] + bl[3]) / 2.0
    xa_n = jnp.concatenate([m[0], m[1], m[5], xa], axis=1) @ Wa + ba
    xu_n = jnp.concatenate([m[2], m[3], xu], axis=1) @ Wu + bu
    xl_n = jnp.concatenate([m[4], xl], axis=1) @ Wlo + bl[4]
    return jax.nn.relu(xu_n), jax.nn.relu(xa_n), jax.nn.relu(xl_n)


def _to_cg_major(x):
    n = x.shape[0]
    return jnp.transpose(x.reshape(n, NCG, W), (1, 0, 2)).reshape(NCG * n, W)


def kernel(x_user, x_ad, x_location, e_click, e_purchase, e_rev_click,
           e_rev_purchase, e_in, e_contains, Wl1, bl1, Wr1, Wl2, bl2, Wr2,
           Wl3, bl3, Wr3, W1, b1, W2, b2):
    edges = (e_click, e_purchase, e_rev_click, e_rev_purchase, e_in, e_contains)
    esrc, edst = [], []
    for r, e in enumerate(edges):
        ep = _EP[REL_E[r]]
        pad = ep - REL_E[r]
        esrc.append(jnp.concatenate([e[0], jnp.zeros((pad,), jnp.int32)]))
        edst.append(jnp.concatenate(
            [e[1], jnp.full((pad,), SENTINEL, jnp.int32)]))
    slack = E_TOTAL - sum(_EP[REL_E[r]] for r in range(6))
    all_src = jnp.concatenate(esrc + [jnp.zeros((slack,), jnp.int32)])
    all_dst = jnp.concatenate(edst + [jnp.full((slack,), SENTINEL, jnp.int32)])

    cnts = _run_counts(edst)
    offtab = jnp.asarray(_OFFTAB)
    zeros = jnp.zeros((B, W), jnp.float32)

    xu, xa, xl = x_user, x_ad, x_location
    for (Wl, bl, Wr) in ((Wl1, bl1, Wr1), (Wl2, bl2, Wr2), (Wl3, bl3, Wr3)):
        btab = jnp.concatenate(
            [_to_cg_major(xu), _to_cg_major(xa), _to_cg_major(xl)])
        sall = _run_segsum(all_src, all_dst, btab, offtab, zeros)
        xu, xa, xl = _dense_layer(sall, cnts, xu, xa, xl, Wl, bl, Wr)

    xu = jax.nn.relu(xu @ W1.T + b1) @ W2.T + b2
    xa = jax.nn.relu(xa @ W1.T + b1) @ W2.T + b2
    xl = jax.nn.relu(xl @ W1.T + b1) @ W2.T + b2
    return (xu, xa, xl)


# speedup vs baseline: 1.4601x; 1.1490x over previous
import numpy as np

import jax
import jax.numpy as jnp
from jax import lax
from jax.experimental import pallas as pl
from jax.experimental.pallas import tpu as pltpu
from jax.experimental.pallas import tpu_sc as plsc

NU, NA, NL, D, H = 50000, 50000, 10000, 128, 128
E_BIG, E_SMALL = 300000, 100000

W = 16
SW = 16
NCG = D // SW
B = 128
SENTINEL = 1 << 20

REL_SRC = ("u", "u", "a", "a", "a", "l")
REL_DST = ("a", "a", "u", "u", "l", "a")
NDST = {"u": NU, "a": NA, "l": NL}
REL_E = (E_BIG, E_BIG, E_BIG, E_BIG, E_SMALL, E_SMALL)

SC_RELS = ((0, 1, 5), (2, 3, 4))


def _round_up(x, m):
    return (x + m - 1) // m * m


_EP = {e: _round_up(e, 16 * B * 4) for e in (E_BIG, E_SMALL)}
_WSL = {e: p // 16 for e, p in _EP.items()}
EPOFF = (0, _EP[E_BIG], 2 * _EP[E_BIG], 3 * _EP[E_BIG],
         4 * _EP[E_BIG], 4 * _EP[E_BIG] + _EP[E_SMALL])
E_TOTAL = 4 * _EP[E_BIG] + 2 * _EP[E_SMALL] + 16384

TABBASE = {"u": 0, "a": NCG * NU, "l": NCG * (NU + NA)}
TAB_ROWS = NCG * (NU + NA + NL)

SEG_NDS = _round_up(NA + 16, 16 * B)
SEG_NR = SEG_NDS // 16
REGOFF = tuple(r * NCG * SEG_NDS for r in range(6))
S_ROWS = 6 * NCG * SEG_NDS
ACC_ROWS = SEG_NDS + 16

CNT_NDS = tuple(_round_up(NDST[t] + 16, 16) for t in REL_DST)

_mesh = plsc.VectorSubcoreMesh(core_axis_name="c", subcore_axis_name="s")
_cparams = pltpu.CompilerParams(use_tc_tiling_on_sc=False)


def _iota16():
    return lax.iota(jnp.int32, 16)


def _sel3(i8, vals):
    return jnp.where(i8 == 0, jnp.int32(vals[0]),
                     jnp.where(i8 == 1, jnp.int32(vals[1]),
                               jnp.int32(vals[2])))


def _count_body(*refs):
    dsts = refs[0:6]
    ones_h, zeros_h = refs[6], refs[7]
    outs = refs[8:14]
    dvm, idxb, onesb, acc, sem = refs[14:]
    core = lax.axis_index("c")
    s = lax.axis_index("s")
    iota = _iota16()
    pltpu.sync_copy(ones_h, onesb)

    for my_core in (0, 1):
        @pl.when(core == my_core)
        def _():
            for r in SC_RELS[my_core]:
                nds = CNT_NDS[r]
                wsl = _WSL[REL_E[r]]
                nz = (nds + 16) // 16
                pltpu.sync_copy(zeros_h.at[pl.ds(s * nz, nz)],
                                acc.at[pl.ds(s * nz, nz)])
                plsc.subcore_barrier()
                pltpu.sync_copy(dsts[r].at[pl.ds(s * wsl, wsl)],
                                dvm.at[pl.ds(0, wsl)])
                garb = nds + lax.bitwise_and(iota, 7)

                def batch(t, _):
                    for u in range(8):
                        dv = dvm[pl.ds(t * B + 16 * u, 16)]
                        idxb[pl.ds(16 * u, 16)] = jnp.minimum(dv, garb)
                    pltpu.sync_copy(onesb, acc.at[idxb], add=True)
                    return 0
                lax.fori_loop(0, wsl // B, batch, 0)
                plsc.subcore_barrier()
                nr = nds // 16
                pltpu.sync_copy(acc.at[pl.ds(s * nr, nr)],
                                outs[r].at[pl.ds(s * nr, nr)])
                plsc.subcore_barrier()


def _run_counts(dsts):
    out_type = tuple(jax.ShapeDtypeStruct((CNT_NDS[r], W), jnp.float32)
                     for r in range(6))
    scratch = [
        pltpu.VMEM((_WSL[E_BIG],), jnp.int32),
        pltpu.VMEM((B,), jnp.int32),
        pltpu.VMEM((B, W), jnp.float32),
        pltpu.VMEM_SHARED((max(CNT_NDS) + 32, W), jnp.float32),
        pltpu.SemaphoreType.DMA,
    ]
    ones = jnp.ones((B, W), jnp.float32)
    zeros = jnp.zeros((max(CNT_NDS) + 32, W), jnp.float32)
    fn = pl.kernel(_count_body, out_type=out_type, mesh=_mesh,
                   scratch_types=scratch, compiler_params=_cparams)
    return fn(*dsts, ones, zeros)


def _make_offtab():
    t = np.zeros((6 * NCG, 2, 16), np.int32)
    for r in range(6):
        nsrc = NDST[REL_SRC[r]]
        for cg in range(NCG):
            t[r * NCG + cg, 0, :] = TABBASE[REL_SRC[r]] + cg * nsrc
            t[r * NCG + cg, 1, :] = SEG_NDS + (np.arange(16) & 7)
    return t


_OFFTAB = _make_offtab()


def _segsum_body(asrc, adst, btab, offtab_h, zeros_h, sout, svm, dvm, offb,
                 i0b, i1b, i2b, i3b, d0b, d1b, d2b, d3b, r0b, r1b, r2b, r3b,
                 g0s, g1s, g2s, g3s, s0s, s1s, s2s, s3s, acc_ref):
    idxbs = (i0b, i1b, i2b, i3b)
    dstbs = (d0b, d1b, d2b, d3b)
    rows = (r0b, r1b, r2b, r3b)
    gsems = (g0s, g1s, g2s, g3s)
    ssems = (s0s, s1s, s2s, s3s)
    core = lax.axis_index("c")
    s = lax.axis_index("s")

    for my_core in (0, 1):
        rels = SC_RELS[my_core]

        @pl.when(core == my_core)
        def _():
            def pass_body(i, _):
                i8 = lax.div(i, NCG)
                cg = lax.rem(i, NCG)
                epoff = _sel3(i8, [EPOFF[r] for r in rels])
                ng = _sel3(i8, [_WSL[REL_E[r]] // B // 4 for r in rels])
                regoff = _sel3(i8, [REGOFF[r] for r in rels])
                offrow = _sel3(i8, [r * NCG for r in rels]) + cg
                wslb = _WSL[E_BIG]

                @pl.when(cg == 0)
                def _():
                    woff = epoff + s * _sel3(i8, [_WSL[REL_E[r]] for r in rels])
                    pltpu.sync_copy(asrc.at[pl.ds(woff, wslb)],
                                    svm.at[pl.ds(0, wslb)])
                    pltpu.sync_copy(adst.at[pl.ds(woff, wslb)],
                                    dvm.at[pl.ds(0, wslb)])

                pltpu.sync_copy(offtab_h.at[offrow], offb)
                offv = offb[0]
                garbv = offb[1]

                pltpu.sync_copy(zeros_h, acc_ref.at[pl.ds(s * SEG_NR, SEG_NR)])
                plsc.subcore_barrier()

                def build(b, t):
                    for u in range(8):
                        base = t * B + 16 * u
                        sv = svm[pl.ds(base, 16)]
                        dv = dvm[pl.ds(base, 16)]
                        idxbs[b][pl.ds(16 * u, 16)] = sv + offv
                        dstbs[b][pl.ds(16 * u, 16)] = jnp.minimum(dv, garbv)

                for b in range(4):
                    build(b, jnp.int32(b))
                    pltpu.async_copy(btab.at[idxbs[b]], rows[b], gsems[b])
                for b in range(4):
                    pltpu.make_async_copy(btab.at[idxbs[b]], rows[b],
                                          gsems[b]).wait()
                    pltpu.async_copy(rows[b], acc_ref.at[dstbs[b]], ssems[b],
                                     add=True)

                def grp(k, _):
                    for b in range(4):
                        pltpu.make_async_copy(rows[b], acc_ref.at[dstbs[b]],
                                              ssems[b]).wait()
                        build(b, k * 4 + b)
                        pltpu.async_copy(btab.at[idxbs[b]], rows[b], gsems[b])
                    for b in range(4):
                        pltpu.make_async_copy(btab.at[idxbs[b]], rows[b],
                                              gsems[b]).wait()
                        pltpu.async_copy(rows[b], acc_ref.at[dstbs[b]],
                                         ssems[b], add=True)
                    return 0
                lax.fori_loop(1, ng, grp, 0)
                for b in range(4):
                    pltpu.make_async_copy(rows[b], acc_ref.at[dstbs[b]],
                                          ssems[b]).wait()
                plsc.subcore_barrier()

                fbase = regoff + cg * SEG_NDS + s * SEG_NR
                pltpu.sync_copy(acc_ref.at[pl.ds(s * SEG_NR, SEG_NR)],
                                sout.at[pl.ds(fbase, SEG_NR)])
                plsc.subcore_barrier()
                return 0

            lax.fori_loop(0, 3 * NCG, pass_body, 0)


def _run_segsum(all_src, all_dst, btab, offtab, zeros):
    out_type = jax.ShapeDtypeStruct((S_ROWS, SW), jnp.float32)
    scratch = (
        [pltpu.VMEM((_WSL[E_BIG],), jnp.int32)] * 2
        + [pltpu.VMEM((2, 16), jnp.int32)]
        + [pltpu.VMEM((B,), jnp.int32) for _ in range(8)]
        + [pltpu.VMEM((B, SW), jnp.float32) for _ in range(4)]
        + [pltpu.SemaphoreType.DMA for _ in range(8)]
        + [pltpu.VMEM_SHARED((ACC_ROWS, SW), jnp.float32)]
    )
    fn = pl.kernel(_segsum_body, out_type=out_type, mesh=_mesh,
                   scratch_types=scratch, compiler_params=_cparams)
    return fn(all_src, all_dst, btab, offtab, zeros)


def _dense_layer(sall, cnts, xu, xa, xl, Wl, bl, Wr):
    m = []
    for r in range(6):
        n = NDST[REL_DST[r]]
        s4 = lax.dynamic_slice_in_dim(sall, REGOFF[r], NCG * SEG_NDS, 0)
        s_full = jnp.transpose(s4.reshape(NCG, SEG_NDS, SW), (1, 0, 2)
                               ).reshape(SEG_NDS, D)[:n]
        inv = 1.0 / jnp.clip(cnts[r][:n, :1], 1.0)
        m.append(s_full * inv)
    Wa = jnp.concatenate(
        [Wl[0].T, Wl[1].T, Wl[5].T, (Wr[0] + Wr[1] + Wr[5]).T], axis=0) / 3.0
    Wu = jnp.concatenate([Wl[2].T, Wl[3].T, (Wr[2] + Wr[3]).T], axis=0) / 2.0
    Wlo = jnp.concatenate([Wl[4].T, Wr[4].T], axis=0)
    ba = (bl[0] + bl[1] + bl[5]) / 3.0
    bu = (bl[2] + bl[3]) / 2.0
    xa_n = jnp.concatenate([m[0], m[1], m[5], xa], axis=1) @ Wa + ba
    xu_n = jnp.concatenate([m[2], m[3], xu], axis=1) @ Wu + bu
    xl_n = jnp.concatenate([m[4], xl], axis=1) @ Wlo + bl[4]
    return jax.nn.relu(xu_n), jax.nn.relu(xa_n), jax.nn.relu(xl_n)


def _to_cg_major(x):
    n = x.shape[0]
    return jnp.transpose(x.reshape(n, NCG, SW), (1, 0, 2)).reshape(NCG * n, SW)


def kernel(x_user, x_ad, x_location, e_click, e_purchase, e_rev_click,
           e_rev_purchase, e_in, e_contains, Wl1, bl1, Wr1, Wl2, bl2, Wr2,
           Wl3, bl3, Wr3, W1, b1, W2, b2):
    edges = (e_click, e_purchase, e_rev_click, e_rev_purchase, e_in, e_contains)
    esrc, edst = [], []
    for r, e in enumerate(edges):
        ep = _EP[REL_E[r]]
        pad = ep - REL_E[r]
        esrc.append(jnp.concatenate([e[0], jnp.zeros((pad,), jnp.int32)]))
        edst.append(jnp.concatenate(
            [e[1], jnp.full((pad,), SENTINEL, jnp.int32)]))
    slack = E_TOTAL - sum(_EP[REL_E[r]] for r in range(6))
    all_src = jnp.concatenate(esrc + [jnp.zeros((slack,), jnp.int32)])
    all_dst = jnp.concatenate(edst + [jnp.full((slack,), SENTINEL, jnp.int32)])

    cnts = _run_counts(edst)
    offtab = jnp.asarray(_OFFTAB)
    zeros = jnp.zeros((SEG_NR, SW), jnp.float32)

    xu, xa, xl = x_user, x_ad, x_location
    for (Wl, bl, Wr) in ((Wl1, bl1, Wr1), (Wl2, bl2, Wr2), (Wl3, bl3, Wr3)):
        btab = jnp.concatenate(
            [_to_cg_major(xu), _to_cg_major(xa), _to_cg_major(xl)])
        sall = _run_segsum(all_src, all_dst, btab, offtab, zeros)
        xu, xa, xl = _dense_layer(sall, cnts, xu, xa, xl, Wl, bl, Wr)

    xu = jax.nn.relu(xu @ W1.T + b1) @ W2.T + b2
    xa = jax.nn.relu(xa @ W1.T + b1) @ W2.T + b2
    xl = jax.nn.relu(xl @ W1.T + b1) @ W2.T + b2
    return (xu, xa, xl)


# speedup vs baseline: 2.3817x; 1.6311x over previous
import numpy as np

import jax
import jax.numpy as jnp
from jax import lax
from jax.experimental import pallas as pl
from jax.experimental.pallas import tpu as pltpu
from jax.experimental.pallas import tpu_sc as plsc

NU, NA, NL, D, H = 50000, 50000, 10000, 128, 128
E_BIG, E_SMALL = 300000, 100000

W = 16
SW = 16
NCG = D // SW
B = 128
SENTINEL = 1 << 20

REL_SRC = ("u", "u", "a", "a", "a", "l")
REL_DST = ("a", "a", "u", "u", "l", "a")
NDST = {"u": NU, "a": NA, "l": NL}
REL_E = (E_BIG, E_BIG, E_BIG, E_BIG, E_SMALL, E_SMALL)

SC_RELS = ((0, 1, 5), (2, 3, 4))


def _round_up(x, m):
    return (x + m - 1) // m * m


_EP = {e: _round_up(e, 16 * B * 4) for e in (E_BIG, E_SMALL)}
_WSL = {e: p // 16 for e, p in _EP.items()}
EPOFF = (0, _EP[E_BIG], 2 * _EP[E_BIG], 3 * _EP[E_BIG],
         4 * _EP[E_BIG], 4 * _EP[E_BIG] + _EP[E_SMALL])
E_TOTAL = 4 * _EP[E_BIG] + 2 * _EP[E_SMALL] + 16384

TABBASE = {"u": 0, "a": NCG * NU, "l": NCG * (NU + NA)}
TAB_ROWS = NCG * (NU + NA + NL)

SEG_NDS = _round_up(NA + 16, 16 * B)
SEG_NR = SEG_NDS // 16
REGOFF = tuple(r * NCG * SEG_NDS for r in range(6))
S_ROWS = 6 * NCG * SEG_NDS
ACC_ROWS = SEG_NDS + 16

CNT_NDS = tuple(_round_up(NDST[t] + 16, 16) for t in REL_DST)

_mesh = plsc.VectorSubcoreMesh(core_axis_name="c", subcore_axis_name="s")
_cparams = pltpu.CompilerParams(use_tc_tiling_on_sc=False)


def _iota16():
    return lax.iota(jnp.int32, 16)


def _sel3(i8, vals):
    return jnp.where(i8 == 0, jnp.int32(vals[0]),
                     jnp.where(i8 == 1, jnp.int32(vals[1]),
                               jnp.int32(vals[2])))


def _count_body(*refs):
    dsts = refs[0:6]
    ones_h, zeros_h = refs[6], refs[7]
    outs = refs[8:14]
    dvm, idxb, onesb, acc, sem = refs[14:]
    core = lax.axis_index("c")
    s = lax.axis_index("s")
    iota = _iota16()
    pltpu.sync_copy(ones_h, onesb)

    for my_core in (0, 1):
        @pl.when(core == my_core)
        def _():
            for r in SC_RELS[my_core]:
                nds = CNT_NDS[r]
                wsl = _WSL[REL_E[r]]
                nz = (nds + 16) // 16
                pltpu.sync_copy(zeros_h.at[pl.ds(s * nz, nz)],
                                acc.at[pl.ds(s * nz, nz)])
                plsc.subcore_barrier()
                pltpu.sync_copy(dsts[r].at[pl.ds(s * wsl, wsl)],
                                dvm.at[pl.ds(0, wsl)])
                garb = nds + lax.bitwise_and(iota, 7)

                def batch(t, _):
                    for u in range(8):
                        dv = dvm[pl.ds(t * B + 16 * u, 16)]
                        idxb[pl.ds(16 * u, 16)] = jnp.minimum(dv, garb)
                    pltpu.sync_copy(onesb, acc.at[idxb], add=True)
                    return 0
                lax.fori_loop(0, wsl // B, batch, 0)
                plsc.subcore_barrier()
                nr = nds // 16
                pltpu.sync_copy(acc.at[pl.ds(s * nr, nr)],
                                outs[r].at[pl.ds(s * nr, nr)])
                plsc.subcore_barrier()


def _run_counts(dsts):
    out_type = tuple(jax.ShapeDtypeStruct((CNT_NDS[r], W), jnp.float32)
                     for r in range(6))
    scratch = [
        pltpu.VMEM((_WSL[E_BIG],), jnp.int32),
        pltpu.VMEM((B,), jnp.int32),
        pltpu.VMEM((B, W), jnp.float32),
        pltpu.VMEM_SHARED((max(CNT_NDS) + 32, W), jnp.float32),
        pltpu.SemaphoreType.DMA,
    ]
    ones = jnp.ones((B, W), jnp.float32)
    zeros = jnp.zeros((max(CNT_NDS) + 32, W), jnp.float32)
    fn = pl.kernel(_count_body, out_type=out_type, mesh=_mesh,
                   scratch_types=scratch, compiler_params=_cparams)
    return fn(*dsts, ones, zeros)


def _make_offtab():
    t = np.zeros((6 * NCG, 2, 16), np.int32)
    for r in range(6):
        nsrc = NDST[REL_SRC[r]]
        for cg in range(NCG):
            t[r * NCG + cg, 0, :] = TABBASE[REL_SRC[r]] + cg * nsrc
            t[r * NCG + cg, 1, :] = SEG_NDS + (np.arange(16) & 7)
    return t


_OFFTAB = _make_offtab()


def _segsum_body(asrc, adst, btab, offtab_h, zeros_h, sout, svm, dvm, offb,
                 i0b, i1b, i2b, i3b, d0b, d1b, d2b, d3b, r0b, r1b, r2b, r3b,
                 g0s, g1s, g2s, g3s, s0s, s1s, s2s, s3s, acc_ref):
    idxbs = (i0b, i1b, i2b, i3b)
    dstbs = (d0b, d1b, d2b, d3b)
    rows = (r0b, r1b, r2b, r3b)
    gsems = (g0s, g1s, g2s, g3s)
    ssems = (s0s, s1s, s2s, s3s)
    core = lax.axis_index("c")
    s = lax.axis_index("s")

    for my_core in (0, 1):
        rels = SC_RELS[my_core]

        @pl.when(core == my_core)
        def _():
            def pass_body(i, _):
                i8 = lax.div(i, NCG)
                cg = lax.rem(i, NCG)
                epoff = _sel3(i8, [EPOFF[r] for r in rels])
                ng = _sel3(i8, [_WSL[REL_E[r]] // B // 4 for r in rels])
                relidx = _sel3(i8, list(rels))
                offrow = _sel3(i8, [r * NCG for r in rels]) + cg
                wslb = _WSL[E_BIG]

                @pl.when(cg == 0)
                def _():
                    woff = epoff + s * _sel3(i8, [_WSL[REL_E[r]] for r in rels])
                    pltpu.sync_copy(asrc.at[pl.ds(woff, wslb)],
                                    svm.at[pl.ds(0, wslb)])
                    pltpu.sync_copy(adst.at[pl.ds(woff, wslb)],
                                    dvm.at[pl.ds(0, wslb)])

                pltpu.sync_copy(offtab_h.at[offrow], offb)
                offv = offb[0]
                garbv = offb[1]

                pltpu.sync_copy(zeros_h, acc_ref.at[pl.ds(s * SEG_NR, SEG_NR)])
                plsc.subcore_barrier()

                def build(b, t):
                    for u in range(8):
                        base = t * B + 16 * u
                        sv = svm[pl.ds(base, 16)]
                        dv = dvm[pl.ds(base, 16)]
                        idxbs[b][pl.ds(16 * u, 16)] = sv + offv
                        dstbs[b][pl.ds(16 * u, 16)] = jnp.minimum(dv, garbv)

                for b in range(4):
                    build(b, jnp.int32(b))
                    pltpu.async_copy(btab.at[idxbs[b]], rows[b], gsems[b])
                for b in range(4):
                    pltpu.make_async_copy(btab.at[idxbs[b]], rows[b],
                                          gsems[b]).wait()
                    pltpu.async_copy(rows[b], acc_ref.at[dstbs[b]], ssems[b],
                                     add=True)

                def grp(k, _):
                    for b in range(4):
                        pltpu.make_async_copy(rows[b], acc_ref.at[dstbs[b]],
                                              ssems[b]).wait()
                        build(b, k * 4 + b)
                        pltpu.async_copy(btab.at[idxbs[b]], rows[b], gsems[b])
                    for b in range(4):
                        pltpu.make_async_copy(btab.at[idxbs[b]], rows[b],
                                              gsems[b]).wait()
                        pltpu.async_copy(rows[b], acc_ref.at[dstbs[b]],
                                         ssems[b], add=True)
                    return 0
                lax.fori_loop(1, ng, grp, 0)
                for b in range(4):
                    pltpu.make_async_copy(rows[b], acc_ref.at[dstbs[b]],
                                          ssems[b]).wait()
                plsc.subcore_barrier()

                pltpu.sync_copy(acc_ref.at[pl.ds(s * SEG_NR, SEG_NR)],
                                sout.at[relidx, pl.ds(s * SEG_NR, SEG_NR),
                                        pl.ds(cg * SW, SW)])
                plsc.subcore_barrier()
                return 0

            lax.fori_loop(0, 3 * NCG, pass_body, 0)


def _run_segsum(all_src, all_dst, btab, offtab, zeros):
    out_type = jax.ShapeDtypeStruct((6, SEG_NDS, D), jnp.float32)
    scratch = (
        [pltpu.VMEM((_WSL[E_BIG],), jnp.int32)] * 2
        + [pltpu.VMEM((2, 16), jnp.int32)]
        + [pltpu.VMEM((B,), jnp.int32) for _ in range(8)]
        + [pltpu.VMEM((B, SW), jnp.float32) for _ in range(4)]
        + [pltpu.SemaphoreType.DMA for _ in range(8)]
        + [pltpu.VMEM_SHARED((ACC_ROWS, SW), jnp.float32)]
    )
    fn = pl.kernel(_segsum_body, out_type=out_type, mesh=_mesh,
                   scratch_types=scratch, compiler_params=_cparams)
    return fn(all_src, all_dst, btab, offtab, zeros)


def _dense_layer(sall, cnts, xu, xa, xl, Wl, bl, Wr):
    m = []
    for r in range(6):
        n = NDST[REL_DST[r]]
        s_full = sall[r, :n]
        inv = 1.0 / jnp.clip(cnts[r][:n, :1], 1.0)
        m.append(s_full * inv)
    Wa = jnp.concatenate(
        [Wl[0].T, Wl[1].T, Wl[5].T, (Wr[0] + Wr[1] + Wr[5]).T], axis=0) / 3.0
    Wu = jnp.concatenate([Wl[2].T, Wl[3].T, (Wr[2] + Wr[3]).T], axis=0) / 2.0
    Wlo = jnp.concatenate([Wl[4].T, Wr[4].T], axis=0)
    ba = (bl[0] + bl[1] + bl[5]) / 3.0
    bu = (bl[2] + bl[3]) / 2.0
    xa_n = jnp.concatenate([m[0], m[1], m[5], xa], axis=1) @ Wa + ba
    xu_n = jnp.concatenate([m[2], m[3], xu], axis=1) @ Wu + bu
    xl_n = jnp.concatenate([m[4], xl], axis=1) @ Wlo + bl[4]
    return jax.nn.relu(xu_n), jax.nn.relu(xa_n), jax.nn.relu(xl_n)


def _to_cg_major(x):
    n = x.shape[0]
    return jnp.transpose(x.reshape(n, NCG, SW), (1, 0, 2)).reshape(NCG * n, SW)


def kernel(x_user, x_ad, x_location, e_click, e_purchase, e_rev_click,
           e_rev_purchase, e_in, e_contains, Wl1, bl1, Wr1, Wl2, bl2, Wr2,
           Wl3, bl3, Wr3, W1, b1, W2, b2):
    edges = (e_click, e_purchase, e_rev_click, e_rev_purchase, e_in, e_contains)
    esrc, edst = [], []
    for r, e in enumerate(edges):
        ep = _EP[REL_E[r]]
        pad = ep - REL_E[r]
        esrc.append(jnp.concatenate([e[0], jnp.zeros((pad,), jnp.int32)]))
        edst.append(jnp.concatenate(
            [e[1], jnp.full((pad,), SENTINEL, jnp.int32)]))
    slack = E_TOTAL - sum(_EP[REL_E[r]] for r in range(6))
    all_src = jnp.concatenate(esrc + [jnp.zeros((slack,), jnp.int32)])
    all_dst = jnp.concatenate(edst + [jnp.full((slack,), SENTINEL, jnp.int32)])

    cnts = _run_counts(edst)
    offtab = jnp.asarray(_OFFTAB)
    zeros = jnp.zeros((SEG_NR, SW), jnp.float32)

    xu, xa, xl = x_user, x_ad, x_location
    for (Wl, bl, Wr) in ((Wl1, bl1, Wr1), (Wl2, bl2, Wr2), (Wl3, bl3, Wr3)):
        btab = jnp.concatenate(
            [_to_cg_major(xu), _to_cg_major(xa), _to_cg_major(xl)])
        sall = _run_segsum(all_src, all_dst, btab, offtab, zeros)
        xu, xa, xl = _dense_layer(sall, cnts, xu, xa, xl, Wl, bl, Wr)

    xu = jax.nn.relu(xu @ W1.T + b1) @ W2.T + b2
    xa = jax.nn.relu(xa @ W1.T + b1) @ W2.T + b2
    xl = jax.nn.relu(xl @ W1.T + b1) @ W2.T + b2
    return (xu, xa, xl)


# speedup vs baseline: 2.4167x; 1.0147x over previous
import numpy as np

import jax
import jax.numpy as jnp
from jax import lax
from jax.experimental import pallas as pl
from jax.experimental.pallas import tpu as pltpu
from jax.experimental.pallas import tpu_sc as plsc

NU, NA, NL, D, H = 50000, 50000, 10000, 128, 128
E_BIG, E_SMALL = 300000, 100000

W = 16
SW = 16
NCG = D // SW
B = 128
SENTINEL = 1 << 20

REL_SRC = ("u", "u", "a", "a", "a", "l")
REL_DST = ("a", "a", "u", "u", "l", "a")
NDST = {"u": NU, "a": NA, "l": NL}
REL_E = (E_BIG, E_BIG, E_BIG, E_BIG, E_SMALL, E_SMALL)

SC_RELS = ((0, 1, 5), (2, 3, 4))


def _round_up(x, m):
    return (x + m - 1) // m * m


_EP = {e: _round_up(e, 16 * B * 4) for e in (E_BIG, E_SMALL)}
_WSL = {e: p // 16 for e, p in _EP.items()}
EPOFF = (0, _EP[E_BIG], 2 * _EP[E_BIG], 3 * _EP[E_BIG],
         4 * _EP[E_BIG], 4 * _EP[E_BIG] + _EP[E_SMALL])
E_TOTAL = 4 * _EP[E_BIG] + 2 * _EP[E_SMALL] + 16384

TABBASE = {"u": 0, "a": NCG * NU, "l": NCG * (NU + NA)}
TAB_ROWS = NCG * (NU + NA + NL)

SEG_NDS = _round_up(NA + 16, 16 * B)
SEG_NR = SEG_NDS // 16
REGOFF = tuple(r * NCG * SEG_NDS for r in range(6))
S_ROWS = 6 * NCG * SEG_NDS
ACC_ROWS = SEG_NDS + 16

CNT_NDS = tuple(_round_up(NDST[t] + 16, 16) for t in REL_DST)

_mesh = plsc.VectorSubcoreMesh(core_axis_name="c", subcore_axis_name="s")
_cparams = pltpu.CompilerParams(use_tc_tiling_on_sc=False)


def _iota16():
    return lax.iota(jnp.int32, 16)


def _sel3(i8, vals):
    return jnp.where(i8 == 0, jnp.int32(vals[0]),
                     jnp.where(i8 == 1, jnp.int32(vals[1]),
                               jnp.int32(vals[2])))


def _count_body(*refs):
    dsts = refs[0:6]
    ones_h, zeros_h = refs[6], refs[7]
    outs = refs[8:14]
    dvm, idxb, onesb, acc, sem = refs[14:]
    core = lax.axis_index("c")
    s = lax.axis_index("s")
    iota = _iota16()
    pltpu.sync_copy(ones_h, onesb)

    for my_core in (0, 1):
        @pl.when(core == my_core)
        def _():
            for r in SC_RELS[my_core]:
                nds = CNT_NDS[r]
                wsl = _WSL[REL_E[r]]
                nz = (nds + 16) // 16
                pltpu.sync_copy(zeros_h.at[pl.ds(s * nz, nz)],
                                acc.at[pl.ds(s * nz, nz)])
                plsc.subcore_barrier()
                pltpu.sync_copy(dsts[r].at[pl.ds(s * wsl, wsl)],
                                dvm.at[pl.ds(0, wsl)])
                garb = nds + lax.bitwise_and(iota, 7)

                def batch(t, _):
                    for u in range(8):
                        dv = dvm[pl.ds(t * B + 16 * u, 16)]
                        idxb[pl.ds(16 * u, 16)] = jnp.minimum(dv, garb)
                    pltpu.sync_copy(onesb, acc.at[idxb], add=True)
                    return 0
                lax.fori_loop(0, wsl // B, batch, 0)
                plsc.subcore_barrier()
                nr = nds // 16
                pltpu.sync_copy(acc.at[pl.ds(s * nr, nr)],
                                outs[r].at[pl.ds(s * nr, nr)])
                plsc.subcore_barrier()


def _run_counts(dsts):
    out_type = tuple(jax.ShapeDtypeStruct((CNT_NDS[r], W), jnp.float32)
                     for r in range(6))
    scratch = [
        pltpu.VMEM((_WSL[E_BIG],), jnp.int32),
        pltpu.VMEM((B,), jnp.int32),
        pltpu.VMEM((B, W), jnp.float32),
        pltpu.VMEM_SHARED((max(CNT_NDS) + 32, W), jnp.float32),
        pltpu.SemaphoreType.DMA,
    ]
    ones = jnp.ones((B, W), jnp.float32)
    zeros = jnp.zeros((max(CNT_NDS) + 32, W), jnp.float32)
    fn = pl.kernel(_count_body, out_type=out_type, mesh=_mesh,
                   scratch_types=scratch, compiler_params=_cparams)
    return fn(*dsts, ones, zeros)


def _make_offtab():
    t = np.zeros((6 * NCG, 2, 16), np.int32)
    for r in range(6):
        nsrc = NDST[REL_SRC[r]]
        for cg in range(NCG):
            t[r * NCG + cg, 0, :] = TABBASE[REL_SRC[r]] + cg * nsrc
            t[r * NCG + cg, 1, :] = SEG_NDS + (np.arange(16) & 7)
    return t


_OFFTAB = _make_offtab()


def _segsum_body(asrc, adst, btab, zeros_h, sout, svm, dvm, dstall,
                 r0b, r1b, r2b, r3b,
                 g0s, g1s, g2s, g3s, s0s, s1s, s2s, s3s, acc_ref):
    rows = (r0b, r1b, r2b, r3b)
    gsems = (g0s, g1s, g2s, g3s)
    ssems = (s0s, s1s, s2s, s3s)
    core = lax.axis_index("c")
    s = lax.axis_index("s")

    for my_core in (0, 1):
        rels = SC_RELS[my_core]

        @pl.when(core == my_core)
        def _():
            def pass_body(i, _):
                i8 = lax.div(i, NCG)
                cg = lax.rem(i, NCG)
                epoff = _sel3(i8, [EPOFF[r] for r in rels])
                ng = _sel3(i8, [_WSL[REL_E[r]] // B // 4 for r in rels])
                relidx = _sel3(i8, list(rels))
                cgbase = _sel3(i8, [TABBASE[REL_SRC[r]] for r in rels]) \
                    + cg * _sel3(i8, [NDST[REL_SRC[r]] for r in rels])
                btabv = btab.at[pl.ds(cgbase, NU)]
                wslb = _WSL[E_BIG]
                garbv = SEG_NDS + lax.bitwise_and(_iota16(), 7)

                @pl.when(cg == 0)
                def _():
                    woff = epoff + s * _sel3(i8, [_WSL[REL_E[r]] for r in rels])
                    pltpu.sync_copy(asrc.at[pl.ds(woff, wslb)],
                                    svm.at[pl.ds(0, wslb)])
                    pltpu.sync_copy(adst.at[pl.ds(woff, wslb)],
                                    dvm.at[pl.ds(0, wslb)])

                    def pb(t, _):
                        for u in range(8):
                            dv = dvm[pl.ds(t * B + 16 * u, 16)]
                            dstall[t, pl.ds(16 * u, 16)] = jnp.minimum(dv,
                                                                       garbv)
                        return 0
                    lax.fori_loop(0, ng * 4, pb, 0)

                pltpu.sync_copy(zeros_h, acc_ref.at[pl.ds(s * SEG_NR, SEG_NR)])
                plsc.subcore_barrier()

                def gidx(t):
                    return btabv.at[svm.at[pl.ds(t * B, B)]]

                for b in range(4):
                    pltpu.async_copy(gidx(jnp.int32(b)), rows[b], gsems[b])
                for b in range(4):
                    pltpu.make_async_copy(gidx(jnp.int32(b)), rows[b],
                                          gsems[b]).wait()
                    pltpu.async_copy(rows[b], acc_ref.at[dstall.at[b]],
                                     ssems[b], add=True)

                def grp(k, _):
                    for b in range(4):
                        pltpu.make_async_copy(rows[b], acc_ref.at[dstall.at[0]],
                                              ssems[b]).wait()
                        pltpu.async_copy(gidx(k * 4 + b), rows[b], gsems[b])
                    for b in range(4):
                        pltpu.make_async_copy(gidx(jnp.int32(b)), rows[b],
                                              gsems[b]).wait()
                        pltpu.async_copy(rows[b],
                                         acc_ref.at[dstall.at[k * 4 + b]],
                                         ssems[b], add=True)
                    return 0
                lax.fori_loop(1, ng, grp, 0)
                for b in range(4):
                    pltpu.make_async_copy(rows[b], acc_ref.at[dstall.at[0]],
                                          ssems[b]).wait()
                plsc.subcore_barrier()

                pltpu.sync_copy(acc_ref.at[pl.ds(s * SEG_NR, SEG_NR)],
                                sout.at[relidx, pl.ds(s * SEG_NR, SEG_NR),
                                        pl.ds(cg * SW, SW)])
                plsc.subcore_barrier()
                return 0

            lax.fori_loop(0, 3 * NCG, pass_body, 0)


def _run_segsum(all_src, all_dst, btab, zeros):
    out_type = jax.ShapeDtypeStruct((6, SEG_NDS, D), jnp.float32)
    scratch = (
        [pltpu.VMEM((_WSL[E_BIG],), jnp.int32)] * 2
        + [pltpu.VMEM((_WSL[E_BIG] // B, B), jnp.int32)]
        + [pltpu.VMEM((B, SW), jnp.float32) for _ in range(4)]
        + [pltpu.SemaphoreType.DMA for _ in range(8)]
        + [pltpu.VMEM_SHARED((ACC_ROWS, SW), jnp.float32)]
    )
    fn = pl.kernel(_segsum_body, out_type=out_type, mesh=_mesh,
                   scratch_types=scratch, compiler_params=_cparams)
    return fn(all_src, all_dst, btab, zeros)


def _dense_layer(sall, cnts, xu, xa, xl, Wl, bl, Wr):
    m = []
    for r in range(6):
        n = NDST[REL_DST[r]]
        s_full = sall[r, :n]
        inv = 1.0 / jnp.clip(cnts[r][:n, :1], 1.0)
        m.append(s_full * inv)
    Wa = jnp.concatenate(
        [Wl[0].T, Wl[1].T, Wl[5].T, (Wr[0] + Wr[1] + Wr[5]).T], axis=0) / 3.0
    Wu = jnp.concatenate([Wl[2].T, Wl[3].T, (Wr[2] + Wr[3]).T], axis=0) / 2.0
    Wlo = jnp.concatenate([Wl[4].T, Wr[4].T], axis=0)
    ba = (bl[0] + bl[1] + bl[5]) / 3.0
    bu = (bl[2] + bl[3]) / 2.0
    xa_n = jnp.concatenate([m[0], m[1], m[5], xa], axis=1) @ Wa + ba
    xu_n = jnp.concatenate([m[2], m[3], xu], axis=1) @ Wu + bu
    xl_n = jnp.concatenate([m[4], xl], axis=1) @ Wlo + bl[4]
    return jax.nn.relu(xu_n), jax.nn.relu(xa_n), jax.nn.relu(xl_n)


def _to_cg_major(x):
    n = x.shape[0]
    return jnp.transpose(x.reshape(n, NCG, SW), (1, 0, 2)).reshape(NCG * n, SW)


def kernel(x_user, x_ad, x_location, e_click, e_purchase, e_rev_click,
           e_rev_purchase, e_in, e_contains, Wl1, bl1, Wr1, Wl2, bl2, Wr2,
           Wl3, bl3, Wr3, W1, b1, W2, b2):
    edges = (e_click, e_purchase, e_rev_click, e_rev_purchase, e_in, e_contains)
    esrc, edst = [], []
    for r, e in enumerate(edges):
        ep = _EP[REL_E[r]]
        pad = ep - REL_E[r]
        esrc.append(jnp.concatenate([e[0], jnp.zeros((pad,), jnp.int32)]))
        edst.append(jnp.concatenate(
            [e[1], jnp.full((pad,), SENTINEL, jnp.int32)]))
    slack = E_TOTAL - sum(_EP[REL_E[r]] for r in range(6))
    all_src = jnp.concatenate(esrc + [jnp.zeros((slack,), jnp.int32)])
    all_dst = jnp.concatenate(edst + [jnp.full((slack,), SENTINEL, jnp.int32)])

    cnts = _run_counts(edst)
    zeros = jnp.zeros((SEG_NR, SW), jnp.float32)

    xu, xa, xl = x_user, x_ad, x_location
    for (Wl, bl, Wr) in ((Wl1, bl1, Wr1), (Wl2, bl2, Wr2), (Wl3, bl3, Wr3)):
        btab = jnp.concatenate(
            [_to_cg_major(xu), _to_cg_major(xa), _to_cg_major(xl),
             jnp.zeros((NU, SW), jnp.float32)])
        sall = _run_segsum(all_src, all_dst, btab, zeros)
        xu, xa, xl = _dense_layer(sall, cnts, xu, xa, xl, Wl, bl, Wr)

    xu = jax.nn.relu(xu @ W1.T + b1) @ W2.T + b2
    xa = jax.nn.relu(xa @ W1.T + b1) @ W2.T + b2
    xl = jax.nn.relu(xl @ W1.T + b1) @ W2.T + b2
    return (xu, xa, xl)
